# Initial kernel scaffold; baseline (speedup 1.0000x reference)
import functools

import jax
import jax.numpy as jnp
from jax import lax
from jax.experimental import pallas as pl
from jax.experimental.pallas import tpu as pltpu
from jax.experimental.pallas import tpu_sc as plsc

N = 50000
E = 1600000
G = 64

KSB = 16
GRPE = KSB * 128
EPAD = 1638400
EB = EPAD // 128
SPAD = 50256
ZPT = SPAD // 16
OPT = N // 16


@functools.lru_cache(maxsize=None)
def _build_agg(nchunks, F, split):
    mesh = plsc.VectorSubcoreMesh(core_axis_name="c", subcore_axis_name="s")
    n_out = 2 if split else nchunks
    out_type = tuple(jax.ShapeDtypeStruct((N, F), jnp.float32)
                     for _ in range(n_out))
    scratch = [
        pltpu.VMEM_SHARED((SPAD, F), jnp.float32),
        pltpu.VMEM((KSB, 128), jnp.int32),
        pltpu.VMEM((KSB, 128), jnp.int32),
        pltpu.VMEM((KSB, 128, F), jnp.float32),
        pltpu.SemaphoreType.DMA,
        pltpu.SemaphoreType.DMA,
    ]

    @functools.partial(pl.kernel, out_type=out_type, mesh=mesh,
                       scratch_types=scratch)
    def agg(src_hbm, dst_hbm, zero_hbm, *rest):
        xs = rest[:nchunks]
        outs = rest[nchunks:nchunks + n_out]
        spmem, src_v, dst_v, rbuf, gsem, ssem = rest[nchunks + n_out:]
        cid = lax.axis_index("c")
        sid = lax.axis_index("s")

        def run(x_ref, out_ref, base_sb, n_groups):
            pltpu.sync_copy(zero_hbm, spmem.at[pl.ds(sid * ZPT, ZPT)])
            plsc.subcore_barrier()

            def body(g, carry):
                sb = base_sb + g * KSB
                pltpu.sync_copy(src_hbm.at[pl.ds(sb, KSB)], src_v)
                pltpu.sync_copy(dst_hbm.at[pl.ds(sb, KSB)], dst_v)
                gs = [pltpu.async_copy(x_ref.at[src_v.at[j]], rbuf.at[j],
                                       gsem) for j in range(KSB)]
                for d in gs:
                    d.wait()
                ss = [pltpu.async_copy(rbuf.at[j], spmem.at[dst_v.at[j]],
                                       ssem, add=True) for j in range(KSB)]
                for d in ss:
                    d.wait()
                return carry

            lax.fori_loop(0, n_groups, body, 0)
            plsc.subcore_barrier()
            pltpu.sync_copy(spmem.at[pl.ds(sid * OPT, OPT)],
                            out_ref.at[pl.ds(sid * OPT, OPT)])
            plsc.subcore_barrier()

        if split:
            wid = cid * 16 + sid
            sb_per_w = EB // 32
            for c in range(2):
                @pl.when(cid == c)
                def _(c=c):
                    run(xs[0], outs[c], wid * sb_per_w, sb_per_w // KSB)
        else:
            sb_per_t = EB // 16
            for p in range(nchunks // 2):
                for c in range(2):
                    @pl.when(cid == c)
                    def _(p=p, c=c):
                        k = 2 * p + c
                        run(xs[k], outs[k], sid * sb_per_t, sb_per_t // KSB)

    return agg


def _sc_agg(src, dst, zero, xs, nchunks, F, split):
    fn = _build_agg(nchunks, F, bool(split))
    return fn(src, dst, zero, *xs)


NB = 2000
NBLK = N // NB


def _row_spec(d):
    return pl.BlockSpec((NB, d), lambda i: (i, 0))


def _full_spec(shape):
    nd = len(shape)
    return pl.BlockSpec(shape, lambda i: (0,) * nd)


def _acc_stats(u, st_ref):
    i = pl.program_id(0)
    s = jnp.sum(u, axis=0, keepdims=True)
    s2 = jnp.sum(u * u, axis=0, keepdims=True)

    @pl.when(i == 0)
    def _():
        st_ref[0:1, :] = s
        st_ref[1:2, :] = s2

    @pl.when(i != 0)
    def _():
        st_ref[0:1, :] = st_ref[0:1, :] + s
        st_ref[1:2, :] = st_ref[1:2, :] + s2


def _bnrelu(u, st, g, be):
    m = st[0:1, :] / N
    var = st[1:2, :] / N - m * m
    rs = lax.rsqrt(var + 1e-5)
    return jnp.maximum((u - m) * (rs * g) + be, 0.0)


def _mm(a, w):
    return jnp.dot(a, w, preferred_element_type=jnp.float32)


def _tc_mm1_l1(x16, p0, p1, w, b):
    do = w.shape[1]

    def body(x_ref, a_ref, b_ref, w_ref, bias_ref, u_ref, st_ref):
        pre = x_ref[...] + a_ref[...] + b_ref[...]
        u = _mm(pre, w_ref[...]) + bias_ref[...]
        u_ref[...] = u
        _acc_stats(u, st_ref)

    return pl.pallas_call(
        body, grid=(NBLK,),
        in_specs=[_row_spec(16), _row_spec(16), _row_spec(16),
                  _full_spec(w.shape), _full_spec(b.shape)],
        out_specs=[_row_spec(do), _full_spec((2, do))],
        out_shape=[jax.ShapeDtypeStruct((N, do), jnp.float32),
                   jax.ShapeDtypeStruct((2, do), jnp.float32)],
    )(x16, p0, p1, w, b)


def _tc_mm1_chunks(xcs, ags, w, b):
    K = len(xcs)
    do = w.shape[1]

    def body(*refs):
        xc = refs[:K]
        ag = refs[K:2 * K]
        w_ref, bias_ref, u_ref, st_ref = refs[2 * K:]
        pre = jnp.concatenate([xc[k][...] + ag[k][...] for k in range(K)],
                              axis=1)
        u = _mm(pre, w_ref[...]) + bias_ref[...]
        u_ref[...] = u
        _acc_stats(u, st_ref)

    return pl.pallas_call(
        body, grid=(NBLK,),
        in_specs=[_row_spec(32)] * (2 * K) + [_full_spec(w.shape),
                                              _full_spec(b.shape)],
        out_specs=[_row_spec(do), _full_spec((2, do))],
        out_shape=[jax.ShapeDtypeStruct((N, do), jnp.float32),
                   jax.ShapeDtypeStruct((2, do), jnp.float32)],
    )(*xcs, *ags, w, b)


def _tc_mm2(u, st, g, be, w, b):
    dh = u.shape[1]
    do = w.shape[1]

    def body(u_ref, st_ref, g_ref, be_ref, w_ref, bias_ref, v_ref, ost_ref):
        h = _bnrelu(u_ref[...], st_ref[...], g_ref[...], be_ref[...])
        v = _mm(h, w_ref[...]) + bias_ref[...]
        v_ref[...] = v
        _acc_stats(v, ost_ref)

    return pl.pallas_call(
        body, grid=(NBLK,),
        in_specs=[_row_spec(dh), _full_spec((2, dh)), _full_spec((1, dh)),
                  _full_spec((1, dh)), _full_spec(w.shape),
                  _full_spec(b.shape)],
        out_specs=[_row_spec(do), _full_spec((2, do))],
        out_shape=[jax.ShapeDtypeStruct((N, do), jnp.float32),
                   jax.ShapeDtypeStruct((2, do), jnp.float32)],
    )(u, st, g, be, w, b)


def _tc_chunks(v, st, g, be):
    dh = v.shape[1]
    K = dh // 32

    def body(v_ref, st_ref, g_ref, be_ref, *outs):
        h = _bnrelu(v_ref[...], st_ref[...], g_ref[...], be_ref[...])
        for k in range(K):
            outs[k][...] = h[:, k * 32:(k + 1) * 32]

    return pl.pallas_call(
        body, grid=(NBLK,),
        in_specs=[_row_spec(dh), _full_spec((2, dh)), _full_spec((1, dh)),
                  _full_spec((1, dh))],
        out_specs=[_row_spec(32)] * K,
        out_shape=[jax.ShapeDtypeStruct((N, 32), jnp.float32)] * K,
    )(v, st, g, be)


def _tc_resid1(v, st, g, be, x16, rw, rb):
    dh = v.shape[1]

    def body(v_ref, st_ref, g_ref, be_ref, x_ref, rw_ref, rb_ref, o0, o1):
        h = _bnrelu(v_ref[...], st_ref[...], g_ref[...], be_ref[...])
        idp = _mm(x_ref[...], rw_ref[...]) + rb_ref[...]
        x1 = jnp.maximum(h + idp, 0.0)
        o0[...] = x1[:, 0:32]
        o1[...] = x1[:, 32:64]

    return pl.pallas_call(
        body, grid=(NBLK,),
        in_specs=[_row_spec(dh), _full_spec((2, dh)), _full_spec((1, dh)),
                  _full_spec((1, dh)), _row_spec(16), _full_spec(rw.shape),
                  _full_spec(rb.shape)],
        out_specs=[_row_spec(32)] * 2,
        out_shape=[jax.ShapeDtypeStruct((N, 32), jnp.float32)] * 2,
    )(v, st, g, be, x16, rw, rb)


def _tc_final(v, st, g, be, x1c0, x1c1, rw, rb, batch3):
    dh = v.shape[1]

    def body(v_ref, st_ref, g_ref, be_ref, c0_ref, c1_ref, rw_ref, rb_ref,
             bid_ref, out_ref):
        i = pl.program_id(0)
        h = _bnrelu(v_ref[...], st_ref[...], g_ref[...], be_ref[...])
        x1 = jnp.concatenate([c0_ref[...], c1_ref[...]], axis=1)
        idp = _mm(x1, rw_ref[...]) + rb_ref[...]
        x2 = jnp.maximum(h + idp, 0.0)

        @pl.when(i == 0)
        def _():
            out_ref[...] = jnp.full((G, dh), -jnp.inf, jnp.float32)

        bid = bid_ref[0, :, :]
        glo = bid[0, 0]
        ghi = bid[NB - 1, 0]

        def gbody(gg, carry):
            mask = bid == gg
            vals = jnp.where(mask, x2, -jnp.inf)
            cm = jnp.max(vals, axis=0, keepdims=True)
            out_ref[pl.ds(gg, 1), :] = jnp.maximum(out_ref[pl.ds(gg, 1), :],
                                                   cm)
            return carry

        lax.fori_loop(glo, ghi + 1, gbody, 0)

    return pl.pallas_call(
        body, grid=(NBLK,),
        in_specs=[_row_spec(dh), _full_spec((2, dh)), _full_spec((1, dh)),
                  _full_spec((1, dh)), _row_spec(32), _row_spec(32),
                  _full_spec(rw.shape), _full_spec(rb.shape),
                  pl.BlockSpec((1, NB, 1), lambda i: (i, 0, 0))],
        out_specs=_full_spec((G, dh)),
        out_shape=jax.ShapeDtypeStruct((G, dh), jnp.float32),
    )(v, st, g, be, x1c0, x1c1, rw, rb, batch3)


def _pad_rows(w, rows):
    return jnp.pad(w, ((0, rows - w.shape[0]), (0, 0)))


def kernel(x, edge_index, batch, params):
    p = params
    src = edge_index[0].astype(jnp.int32)
    dst = edge_index[1].astype(jnp.int32)

    pad = EPAD - E
    fill = (jnp.arange(pad, dtype=jnp.int32) % 256)
    src_p = jnp.concatenate([src, fill]).reshape(EB, 128)
    dst_p = jnp.concatenate([dst, N + fill]).reshape(EB, 128)

    x16 = jnp.pad(x, ((0, 0), (0, 16 - x.shape[1])))
    zero16 = jnp.zeros((ZPT, 16), jnp.float32)
    zero32 = jnp.zeros((ZPT, 32), jnp.float32)
    batch3 = batch.astype(jnp.int32).reshape(NBLK, NB, 1)

    def v2(a):
        return a.reshape(1, -1)

    a0, a1 = _sc_agg(src_p, dst_p, zero16, (x16,), 1, 16, True)
    u, st = _tc_mm1_l1(x16, a0, a1, _pad_rows(p["c11"]["w1"], 16),
                       v2(p["c11"]["b1"]))
    v, st = _tc_mm2(u, st, v2(p["c11"]["g1"]), v2(p["c11"]["be1"]),
                    p["c11"]["w2"], v2(p["c11"]["b2"]))
    oc = _tc_chunks(v, st, v2(p["c11"]["g2"]), v2(p["c11"]["be2"]))

    ag = _sc_agg(src_p, dst_p, zero32, tuple(oc), 2, 32, False)
    u, st = _tc_mm1_chunks(oc, ag, p["c12"]["w1"], v2(p["c12"]["b1"]))
    v, st = _tc_mm2(u, st, v2(p["c12"]["g1"]), v2(p["c12"]["be1"]),
                    p["c12"]["w2"], v2(p["c12"]["b2"]))
    x1c = _tc_resid1(v, st, v2(p["c12"]["g2"]), v2(p["c12"]["be2"]), x16,
                     _pad_rows(p["r1"]["w"], 16), v2(p["r1"]["b"]))

    ag = _sc_agg(src_p, dst_p, zero32, tuple(x1c), 2, 32, False)
    u, st = _tc_mm1_chunks(x1c, ag, p["c21"]["w1"], v2(p["c21"]["b1"]))
    v, st = _tc_mm2(u, st, v2(p["c21"]["g1"]), v2(p["c21"]["be1"]),
                    p["c21"]["w2"], v2(p["c21"]["b2"]))
    oc = _tc_chunks(v, st, v2(p["c21"]["g2"]), v2(p["c21"]["be2"]))

    ag = _sc_agg(src_p, dst_p, zero32, tuple(oc), 4, 32, False)
    u, st = _tc_mm1_chunks(oc, ag, p["c22"]["w1"], v2(p["c22"]["b1"]))
    v, st = _tc_mm2(u, st, v2(p["c22"]["g1"]), v2(p["c22"]["be1"]),
                    p["c22"]["w2"], v2(p["c22"]["b2"]))

    return _tc_final(v, st, v2(p["c22"]["g2"]), v2(p["c22"]["be2"]),
                     x1c[0], x1c[1], p["r2"]["w"], v2(p["r2"]["b"]), batch3)


# speedup vs baseline: 10.4983x; 10.4983x over previous
import functools

import jax
import jax.numpy as jnp
from jax import lax
from jax.experimental import pallas as pl
from jax.experimental.pallas import tpu as pltpu
from jax.experimental.pallas import tpu_sc as plsc

N = 50000
E = 1600000
G = 64

KSB = 16
GRPE = KSB * 128
BATCH = 2
NBATCH = KSB // BATCH
EPAD = 1638400
EB = EPAD // 128
SPAD = 50304
ZPT = SPAD // 16
OPT = 3128


@functools.lru_cache(maxsize=None)
def _build_agg(nchunks, F, split):
    mesh = plsc.VectorSubcoreMesh(core_axis_name="c", subcore_axis_name="s")
    n_out = 2 if split else nchunks
    out_type = tuple(jax.ShapeDtypeStruct((N, F), jnp.float32)
                     for _ in range(n_out))
    scratch = [
        pltpu.VMEM_SHARED((SPAD, F), jnp.float32),
        pltpu.VMEM((KSB, 128), jnp.int32),
        pltpu.VMEM((KSB, 128), jnp.int32),
        pltpu.VMEM((2, BATCH, 128, F), jnp.float32),
        pltpu.SemaphoreType.DMA,
        pltpu.SemaphoreType.DMA,
    ]

    @functools.partial(
        pl.kernel, out_type=out_type, mesh=mesh, scratch_types=scratch,
        compiler_params=pltpu.CompilerParams(use_tc_tiling_on_sc=False))
    def agg(src_hbm, dst_hbm, zero_hbm, *rest):
        xs = rest[:nchunks]
        outs = rest[nchunks:nchunks + n_out]
        spmem, src_v, dst_v, rbuf, gsem, ssem = rest[nchunks + n_out:]
        cid = lax.axis_index("c")
        sid = lax.axis_index("s")

        def run(x_ref, out_ref, base_sb, n_groups):
            pltpu.sync_copy(zero_hbm, spmem.at[pl.ds(sid * ZPT, ZPT)])
            plsc.subcore_barrier()

            def body(g, carry):
                sb = base_sb + g * KSB
                pltpu.sync_copy(src_hbm.at[pl.ds(sb, KSB)], src_v)
                pltpu.sync_copy(dst_hbm.at[pl.ds(sb, KSB)], dst_v)

                def gather(b):
                    return [pltpu.async_copy(
                        x_ref.at[src_v.at[BATCH * b + t]],
                        rbuf.at[b % 2, t], gsem) for t in range(BATCH)]

                def scatter(b):
                    return [pltpu.async_copy(
                        rbuf.at[b % 2, t],
                        spmem.at[dst_v.at[BATCH * b + t]], ssem, add=True)
                        for t in range(BATCH)]

                gd = {0: gather(0)}
                sd = {}
                for b in range(1, NBATCH):
                    if b >= 2:
                        for d2 in sd[b - 2]:
                            d2.wait()
                    gd[b] = gather(b)
                    for d2 in gd[b - 1]:
                        d2.wait()
                    sd[b - 1] = scatter(b - 1)
                for d2 in gd[NBATCH - 1]:
                    d2.wait()
                sd[NBATCH - 1] = scatter(NBATCH - 1)
                for d2 in sd[NBATCH - 2]:
                    d2.wait()
                for d2 in sd[NBATCH - 1]:
                    d2.wait()
                return carry

            lax.fori_loop(0, n_groups, body, 0)
            plsc.subcore_barrier()

            @pl.when(sid < 15)
            def _():
                pltpu.sync_copy(spmem.at[pl.ds(sid * OPT, OPT)],
                                out_ref.at[pl.ds(sid * OPT, OPT)])

            @pl.when(sid == 15)
            def _():
                pltpu.sync_copy(spmem.at[pl.ds(15 * OPT, N - 15 * OPT)],
                                out_ref.at[pl.ds(15 * OPT, N - 15 * OPT)])

            plsc.subcore_barrier()

        if split:
            wid = cid * 16 + sid
            sb_per_w = EB // 32
            for c in range(2):
                @pl.when(cid == c)
                def _(c=c):
                    run(xs[0], outs[c], wid * sb_per_w, sb_per_w // KSB)
        else:
            sb_per_t = EB // 16
            for p in range(nchunks // 2):
                for c in range(2):
                    @pl.when(cid == c)
                    def _(p=p, c=c):
                        k = 2 * p + c
                        run(xs[k], outs[k], sid * sb_per_t, sb_per_t // KSB)

    return agg


def _sc_agg(src, dst, zero, xs, nchunks, F, split):
    fn = _build_agg(nchunks, F, bool(split))
    return fn(src, dst, zero, *xs)


NB = 2000
NBLK = N // NB


def _row_spec(d):
    return pl.BlockSpec((NB, d), lambda i: (i, 0))


def _full_spec(shape):
    nd = len(shape)
    return pl.BlockSpec(shape, lambda i: (0,) * nd)


def _acc_stats(u, st_ref):
    i = pl.program_id(0)
    s = jnp.sum(u, axis=0, keepdims=True)
    s2 = jnp.sum(u * u, axis=0, keepdims=True)

    @pl.when(i == 0)
    def _():
        st_ref[0:1, :] = s
        st_ref[1:2, :] = s2

    @pl.when(i != 0)
    def _():
        st_ref[0:1, :] = st_ref[0:1, :] + s
        st_ref[1:2, :] = st_ref[1:2, :] + s2


def _bnrelu(u, st, g, be):
    m = st[0:1, :] / N
    var = st[1:2, :] / N - m * m
    rs = lax.rsqrt(var + 1e-5)
    return jnp.maximum((u - m) * (rs * g) + be, 0.0)


def _mm(a, w):
    return jnp.dot(a, w, preferred_element_type=jnp.float32)


def _tc_mm1_l1(x16, p0, p1, w, b):
    do = w.shape[1]

    def body(x_ref, a_ref, b_ref, w_ref, bias_ref, u_ref, st_ref):
        pre = x_ref[...] + a_ref[...] + b_ref[...]
        u = _mm(pre, w_ref[...]) + bias_ref[...]
        u_ref[...] = u
        _acc_stats(u, st_ref)

    return pl.pallas_call(
        body, grid=(NBLK,),
        in_specs=[_row_spec(16), _row_spec(16), _row_spec(16),
                  _full_spec(w.shape), _full_spec(b.shape)],
        out_specs=[_row_spec(do), _full_spec((2, do))],
        out_shape=[jax.ShapeDtypeStruct((N, do), jnp.float32),
                   jax.ShapeDtypeStruct((2, do), jnp.float32)],
    )(x16, p0, p1, w, b)


def _tc_mm1_chunks(xcs, ags, w, b):
    K = len(xcs)
    do = w.shape[1]

    def body(*refs):
        xc = refs[:K]
        ag = refs[K:2 * K]
        w_ref, bias_ref, u_ref, st_ref = refs[2 * K:]
        pre = jnp.concatenate([xc[k][...] + ag[k][...] for k in range(K)],
                              axis=1)
        u = _mm(pre, w_ref[...]) + bias_ref[...]
        u_ref[...] = u
        _acc_stats(u, st_ref)

    return pl.pallas_call(
        body, grid=(NBLK,),
        in_specs=[_row_spec(32)] * (2 * K) + [_full_spec(w.shape),
                                              _full_spec(b.shape)],
        out_specs=[_row_spec(do), _full_spec((2, do))],
        out_shape=[jax.ShapeDtypeStruct((N, do), jnp.float32),
                   jax.ShapeDtypeStruct((2, do), jnp.float32)],
    )(*xcs, *ags, w, b)


def _tc_mm2(u, st, g, be, w, b):
    dh = u.shape[1]
    do = w.shape[1]

    def body(u_ref, st_ref, g_ref, be_ref, w_ref, bias_ref, v_ref, ost_ref):
        h = _bnrelu(u_ref[...], st_ref[...], g_ref[...], be_ref[...])
        v = _mm(h, w_ref[...]) + bias_ref[...]
        v_ref[...] = v
        _acc_stats(v, ost_ref)

    return pl.pallas_call(
        body, grid=(NBLK,),
        in_specs=[_row_spec(dh), _full_spec((2, dh)), _full_spec((1, dh)),
                  _full_spec((1, dh)), _full_spec(w.shape),
                  _full_spec(b.shape)],
        out_specs=[_row_spec(do), _full_spec((2, do))],
        out_shape=[jax.ShapeDtypeStruct((N, do), jnp.float32),
                   jax.ShapeDtypeStruct((2, do), jnp.float32)],
    )(u, st, g, be, w, b)


def _tc_chunks(v, st, g, be):
    dh = v.shape[1]
    K = dh // 32

    def body(v_ref, st_ref, g_ref, be_ref, *outs):
        h = _bnrelu(v_ref[...], st_ref[...], g_ref[...], be_ref[...])
        for k in range(K):
            outs[k][...] = h[:, k * 32:(k + 1) * 32]

    return pl.pallas_call(
        body, grid=(NBLK,),
        in_specs=[_row_spec(dh), _full_spec((2, dh)), _full_spec((1, dh)),
                  _full_spec((1, dh))],
        out_specs=[_row_spec(32)] * K,
        out_shape=[jax.ShapeDtypeStruct((N, 32), jnp.float32)] * K,
    )(v, st, g, be)


def _tc_resid1(v, st, g, be, x16, rw, rb):
    dh = v.shape[1]

    def body(v_ref, st_ref, g_ref, be_ref, x_ref, rw_ref, rb_ref, o0, o1):
        h = _bnrelu(v_ref[...], st_ref[...], g_ref[...], be_ref[...])
        idp = _mm(x_ref[...], rw_ref[...]) + rb_ref[...]
        x1 = jnp.maximum(h + idp, 0.0)
        o0[...] = x1[:, 0:32]
        o1[...] = x1[:, 32:64]

    return pl.pallas_call(
        body, grid=(NBLK,),
        in_specs=[_row_spec(dh), _full_spec((2, dh)), _full_spec((1, dh)),
                  _full_spec((1, dh)), _row_spec(16), _full_spec(rw.shape),
                  _full_spec(rb.shape)],
        out_specs=[_row_spec(32)] * 2,
        out_shape=[jax.ShapeDtypeStruct((N, 32), jnp.float32)] * 2,
    )(v, st, g, be, x16, rw, rb)


def _tc_final(v, st, g, be, x1c0, x1c1, rw, rb, batch3):
    dh = v.shape[1]

    def body(v_ref, st_ref, g_ref, be_ref, c0_ref, c1_ref, rw_ref, rb_ref,
             bid_ref, out_ref):
        i = pl.program_id(0)
        h = _bnrelu(v_ref[...], st_ref[...], g_ref[...], be_ref[...])
        x1 = jnp.concatenate([c0_ref[...], c1_ref[...]], axis=1)
        idp = _mm(x1, rw_ref[...]) + rb_ref[...]
        x2 = jnp.maximum(h + idp, 0.0)

        @pl.when(i == 0)
        def _():
            out_ref[...] = jnp.full((G, dh), -jnp.inf, jnp.float32)

        bid = bid_ref[0, :, :]
        glo = bid[0, 0]
        ghi = bid[NB - 1, 0]

        def gbody(gg, carry):
            mask = bid == gg
            vals = jnp.where(mask, x2, -jnp.inf)
            cm = jnp.max(vals, axis=0, keepdims=True)
            out_ref[pl.ds(gg, 1), :] = jnp.maximum(out_ref[pl.ds(gg, 1), :],
                                                   cm)
            return carry

        lax.fori_loop(glo, ghi + 1, gbody, 0)

    return pl.pallas_call(
        body, grid=(NBLK,),
        in_specs=[_row_spec(dh), _full_spec((2, dh)), _full_spec((1, dh)),
                  _full_spec((1, dh)), _row_spec(32), _row_spec(32),
                  _full_spec(rw.shape), _full_spec(rb.shape),
                  pl.BlockSpec((1, NB, 1), lambda i: (i, 0, 0))],
        out_specs=_full_spec((G, dh)),
        out_shape=jax.ShapeDtypeStruct((G, dh), jnp.float32),
    )(v, st, g, be, x1c0, x1c1, rw, rb, batch3)


def _pad_rows(w, rows):
    return jnp.pad(w, ((0, rows - w.shape[0]), (0, 0)))


def kernel(x, edge_index, batch, params):
    p = params
    src = edge_index[0].astype(jnp.int32)
    dst = edge_index[1].astype(jnp.int32)

    pad = EPAD - E
    fill = (jnp.arange(pad, dtype=jnp.int32) % 256)
    src_p = jnp.concatenate([src, fill]).reshape(EB, 128)
    dst_p = jnp.concatenate([dst, N + fill]).reshape(EB, 128)

    x16 = jnp.pad(x, ((0, 0), (0, 16 - x.shape[1])))
    zero16 = jnp.zeros((ZPT, 16), jnp.float32)
    zero32 = jnp.zeros((ZPT, 32), jnp.float32)
    batch3 = batch.astype(jnp.int32).reshape(NBLK, NB, 1)

    def v2(a):
        return a.reshape(1, -1)

    a0, a1 = _sc_agg(src_p, dst_p, zero16, (x16,), 1, 16, True)
    u, st = _tc_mm1_l1(x16, a0, a1, _pad_rows(p["c11"]["w1"], 16),
                       v2(p["c11"]["b1"]))
    v, st = _tc_mm2(u, st, v2(p["c11"]["g1"]), v2(p["c11"]["be1"]),
                    p["c11"]["w2"], v2(p["c11"]["b2"]))
    oc = _tc_chunks(v, st, v2(p["c11"]["g2"]), v2(p["c11"]["be2"]))

    ag = _sc_agg(src_p, dst_p, zero32, tuple(oc), 2, 32, False)
    u, st = _tc_mm1_chunks(oc, ag, p["c12"]["w1"], v2(p["c12"]["b1"]))
    v, st = _tc_mm2(u, st, v2(p["c12"]["g1"]), v2(p["c12"]["be1"]),
                    p["c12"]["w2"], v2(p["c12"]["b2"]))
    x1c = _tc_resid1(v, st, v2(p["c12"]["g2"]), v2(p["c12"]["be2"]), x16,
                     _pad_rows(p["r1"]["w"], 16), v2(p["r1"]["b"]))

    ag = _sc_agg(src_p, dst_p, zero32, tuple(x1c), 2, 32, False)
    u, st = _tc_mm1_chunks(x1c, ag, p["c21"]["w1"], v2(p["c21"]["b1"]))
    v, st = _tc_mm2(u, st, v2(p["c21"]["g1"]), v2(p["c21"]["be1"]),
                    p["c21"]["w2"], v2(p["c21"]["b2"]))
    oc = _tc_chunks(v, st, v2(p["c21"]["g2"]), v2(p["c21"]["be2"]))

    ag = _sc_agg(src_p, dst_p, zero32, tuple(oc), 4, 32, False)
    u, st = _tc_mm1_chunks(oc, ag, p["c22"]["w1"], v2(p["c22"]["b1"]))
    v, st = _tc_mm2(u, st, v2(p["c22"]["g1"]), v2(p["c22"]["be1"]),
                    p["c22"]["w2"], v2(p["c22"]["b2"]))

    return _tc_final(v, st, v2(p["c22"]["g2"]), v2(p["c22"]["be2"]),
                     x1c[0], x1c[1], p["r2"]["w"], v2(p["r2"]["b"]), batch3)


# speedup vs baseline: 11.7418x; 1.1184x over previous
import functools

import jax
import jax.numpy as jnp
from jax import lax
from jax.experimental import pallas as pl
from jax.experimental.pallas import tpu as pltpu
from jax.experimental.pallas import tpu_sc as plsc

N = 50000
E = 1600000
G = 64

KSB = 16
GRPE = KSB * 128
RING = 6
LAG = 3
EPAD = 1638400
EB = EPAD // 128
SPAD = 50304
ZPT = SPAD // 16
OPT = 3128


@functools.lru_cache(maxsize=None)
def _build_agg(nchunks, F, split):
    mesh = plsc.VectorSubcoreMesh(core_axis_name="c", subcore_axis_name="s")
    n_out = 2 if split else nchunks
    out_type = tuple(jax.ShapeDtypeStruct((N, F), jnp.float32)
                     for _ in range(n_out))
    scratch = [
        pltpu.VMEM_SHARED((SPAD, F), jnp.float32),
        pltpu.VMEM((KSB, 128), jnp.int32),
        pltpu.VMEM((KSB, 128), jnp.int32),
        pltpu.VMEM((RING, 128, F), jnp.float32),
        pltpu.SemaphoreType.DMA,
        pltpu.SemaphoreType.DMA,
    ]

    @functools.partial(
        pl.kernel, out_type=out_type, mesh=mesh, scratch_types=scratch,
        compiler_params=pltpu.CompilerParams(use_tc_tiling_on_sc=False))
    def agg(src_hbm, dst_hbm, zero_hbm, *rest):
        xs = rest[:nchunks]
        outs = rest[nchunks:nchunks + n_out]
        spmem, src_v, dst_v, rbuf, gsem, ssem = rest[nchunks + n_out:]
        cid = lax.axis_index("c")
        sid = lax.axis_index("s")

        def run(x_ref, out_ref, base_sb, n_groups):
            pltpu.sync_copy(zero_hbm, spmem.at[pl.ds(sid * ZPT, ZPT)])
            plsc.subcore_barrier()

            def body(g, carry):
                sb = base_sb + g * KSB
                pltpu.sync_copy(src_hbm.at[pl.ds(sb, KSB)], src_v)
                pltpu.sync_copy(dst_hbm.at[pl.ds(sb, KSB)], dst_v)

                def gather(j):
                    return pltpu.async_copy(
                        x_ref.at[src_v.at[j]], rbuf.at[j % RING], gsem)

                def scatter(j):
                    return pltpu.async_copy(
                        rbuf.at[j % RING], spmem.at[dst_v.at[j]], ssem,
                        add=True)

                gd = {}
                sd = {}
                for j in range(KSB):
                    if j >= RING:
                        sd[j - RING].wait()
                    gd[j] = gather(j)
                    if j >= LAG:
                        gd[j - LAG].wait()
                        sd[j - LAG] = scatter(j - LAG)
                for j in range(KSB - LAG, KSB):
                    gd[j].wait()
                    sd[j] = scatter(j)
                for j in range(KSB - RING, KSB):
                    sd[j].wait()
                return carry

            lax.fori_loop(0, n_groups, body, 0)
            plsc.subcore_barrier()

            @pl.when(sid < 15)
            def _():
                pltpu.sync_copy(spmem.at[pl.ds(sid * OPT, OPT)],
                                out_ref.at[pl.ds(sid * OPT, OPT)])

            @pl.when(sid == 15)
            def _():
                pltpu.sync_copy(spmem.at[pl.ds(15 * OPT, N - 15 * OPT)],
                                out_ref.at[pl.ds(15 * OPT, N - 15 * OPT)])

            plsc.subcore_barrier()

        if split:
            wid = cid * 16 + sid
            sb_per_w = EB // 32
            for c in range(2):
                @pl.when(cid == c)
                def _(c=c):
                    run(xs[0], outs[c], wid * sb_per_w, sb_per_w // KSB)
        else:
            sb_per_t = EB // 16
            for p in range(nchunks // 2):
                for c in range(2):
                    @pl.when(cid == c)
                    def _(p=p, c=c):
                        k = 2 * p + c
                        run(xs[k], outs[k], sid * sb_per_t, sb_per_t // KSB)

    return agg


def _sc_agg(src, dst, zero, xs, nchunks, F, split):
    fn = _build_agg(nchunks, F, bool(split))
    return fn(src, dst, zero, *xs)


NB = 2000
NBLK = N // NB


def _row_spec(d):
    return pl.BlockSpec((NB, d), lambda i: (i, 0))


def _full_spec(shape):
    nd = len(shape)
    return pl.BlockSpec(shape, lambda i: (0,) * nd)


def _acc_stats(u, st_ref):
    i = pl.program_id(0)
    s = jnp.sum(u, axis=0, keepdims=True)
    s2 = jnp.sum(u * u, axis=0, keepdims=True)

    @pl.when(i == 0)
    def _():
        st_ref[0:1, :] = s
        st_ref[1:2, :] = s2

    @pl.when(i != 0)
    def _():
        st_ref[0:1, :] = st_ref[0:1, :] + s
        st_ref[1:2, :] = st_ref[1:2, :] + s2


def _bnrelu(u, st, g, be):
    m = st[0:1, :] / N
    var = st[1:2, :] / N - m * m
    rs = lax.rsqrt(var + 1e-5)
    return jnp.maximum((u - m) * (rs * g) + be, 0.0)


def _mm(a, w):
    return jnp.dot(a, w, preferred_element_type=jnp.float32)


def _tc_mm1_l1(x16, p0, p1, w, b):
    do = w.shape[1]

    def body(x_ref, a_ref, b_ref, w_ref, bias_ref, u_ref, st_ref):
        pre = x_ref[...] + a_ref[...] + b_ref[...]
        u = _mm(pre, w_ref[...]) + bias_ref[...]
        u_ref[...] = u
        _acc_stats(u, st_ref)

    return pl.pallas_call(
        body, grid=(NBLK,),
        in_specs=[_row_spec(16), _row_spec(16), _row_spec(16),
                  _full_spec(w.shape), _full_spec(b.shape)],
        out_specs=[_row_spec(do), _full_spec((2, do))],
        out_shape=[jax.ShapeDtypeStruct((N, do), jnp.float32),
                   jax.ShapeDtypeStruct((2, do), jnp.float32)],
    )(x16, p0, p1, w, b)


def _tc_mm1_chunks(xcs, ags, w, b):
    K = len(xcs)
    do = w.shape[1]

    def body(*refs):
        xc = refs[:K]
        ag = refs[K:2 * K]
        w_ref, bias_ref, u_ref, st_ref = refs[2 * K:]
        pre = jnp.concatenate([xc[k][...] + ag[k][...] for k in range(K)],
                              axis=1)
        u = _mm(pre, w_ref[...]) + bias_ref[...]
        u_ref[...] = u
        _acc_stats(u, st_ref)

    return pl.pallas_call(
        body, grid=(NBLK,),
        in_specs=[_row_spec(32)] * (2 * K) + [_full_spec(w.shape),
                                              _full_spec(b.shape)],
        out_specs=[_row_spec(do), _full_spec((2, do))],
        out_shape=[jax.ShapeDtypeStruct((N, do), jnp.float32),
                   jax.ShapeDtypeStruct((2, do), jnp.float32)],
    )(*xcs, *ags, w, b)


def _tc_mm2(u, st, g, be, w, b):
    dh = u.shape[1]
    do = w.shape[1]

    def body(u_ref, st_ref, g_ref, be_ref, w_ref, bias_ref, v_ref, ost_ref):
        h = _bnrelu(u_ref[...], st_ref[...], g_ref[...], be_ref[...])
        v = _mm(h, w_ref[...]) + bias_ref[...]
        v_ref[...] = v
        _acc_stats(v, ost_ref)

    return pl.pallas_call(
        body, grid=(NBLK,),
        in_specs=[_row_spec(dh), _full_spec((2, dh)), _full_spec((1, dh)),
                  _full_spec((1, dh)), _full_spec(w.shape),
                  _full_spec(b.shape)],
        out_specs=[_row_spec(do), _full_spec((2, do))],
        out_shape=[jax.ShapeDtypeStruct((N, do), jnp.float32),
                   jax.ShapeDtypeStruct((2, do), jnp.float32)],
    )(u, st, g, be, w, b)


def _tc_chunks(v, st, g, be):
    dh = v.shape[1]
    K = dh // 32

    def body(v_ref, st_ref, g_ref, be_ref, *outs):
        h = _bnrelu(v_ref[...], st_ref[...], g_ref[...], be_ref[...])
        for k in range(K):
            outs[k][...] = h[:, k * 32:(k + 1) * 32]

    return pl.pallas_call(
        body, grid=(NBLK,),
        in_specs=[_row_spec(dh), _full_spec((2, dh)), _full_spec((1, dh)),
                  _full_spec((1, dh))],
        out_specs=[_row_spec(32)] * K,
        out_shape=[jax.ShapeDtypeStruct((N, 32), jnp.float32)] * K,
    )(v, st, g, be)


def _tc_resid1(v, st, g, be, x16, rw, rb):
    dh = v.shape[1]

    def body(v_ref, st_ref, g_ref, be_ref, x_ref, rw_ref, rb_ref, o0, o1):
        h = _bnrelu(v_ref[...], st_ref[...], g_ref[...], be_ref[...])
        idp = _mm(x_ref[...], rw_ref[...]) + rb_ref[...]
        x1 = jnp.maximum(h + idp, 0.0)
        o0[...] = x1[:, 0:32]
        o1[...] = x1[:, 32:64]

    return pl.pallas_call(
        body, grid=(NBLK,),
        in_specs=[_row_spec(dh), _full_spec((2, dh)), _full_spec((1, dh)),
                  _full_spec((1, dh)), _row_spec(16), _full_spec(rw.shape),
                  _full_spec(rb.shape)],
        out_specs=[_row_spec(32)] * 2,
        out_shape=[jax.ShapeDtypeStruct((N, 32), jnp.float32)] * 2,
    )(v, st, g, be, x16, rw, rb)


def _tc_final(v, st, g, be, x1c0, x1c1, rw, rb, batch3):
    dh = v.shape[1]

    def body(v_ref, st_ref, g_ref, be_ref, c0_ref, c1_ref, rw_ref, rb_ref,
             bid_ref, out_ref):
        i = pl.program_id(0)
        h = _bnrelu(v_ref[...], st_ref[...], g_ref[...], be_ref[...])
        x1 = jnp.concatenate([c0_ref[...], c1_ref[...]], axis=1)
        idp = _mm(x1, rw_ref[...]) + rb_ref[...]
        x2 = jnp.maximum(h + idp, 0.0)

        @pl.when(i == 0)
        def _():
            out_ref[...] = jnp.full((G, dh), -jnp.inf, jnp.float32)

        bid = bid_ref[0, :, :]
        glo = bid[0, 0]
        ghi = bid[NB - 1, 0]

        def gbody(gg, carry):
            mask = bid == gg
            vals = jnp.where(mask, x2, -jnp.inf)
            cm = jnp.max(vals, axis=0, keepdims=True)
            out_ref[pl.ds(gg, 1), :] = jnp.maximum(out_ref[pl.ds(gg, 1), :],
                                                   cm)
            return carry

        lax.fori_loop(glo, ghi + 1, gbody, 0)

    return pl.pallas_call(
        body, grid=(NBLK,),
        in_specs=[_row_spec(dh), _full_spec((2, dh)), _full_spec((1, dh)),
                  _full_spec((1, dh)), _row_spec(32), _row_spec(32),
                  _full_spec(rw.shape), _full_spec(rb.shape),
                  pl.BlockSpec((1, NB, 1), lambda i: (i, 0, 0))],
        out_specs=_full_spec((G, dh)),
        out_shape=jax.ShapeDtypeStruct((G, dh), jnp.float32),
    )(v, st, g, be, x1c0, x1c1, rw, rb, batch3)


def _pad_rows(w, rows):
    return jnp.pad(w, ((0, rows - w.shape[0]), (0, 0)))


def kernel(x, edge_index, batch, params):
    p = params
    src = edge_index[0].astype(jnp.int32)
    dst = edge_index[1].astype(jnp.int32)

    pad = EPAD - E
    fill = (jnp.arange(pad, dtype=jnp.int32) % 256)
    src_p = jnp.concatenate([src, fill]).reshape(EB, 128)
    dst_p = jnp.concatenate([dst, N + fill]).reshape(EB, 128)

    x16 = jnp.pad(x, ((0, 0), (0, 16 - x.shape[1])))
    zero16 = jnp.zeros((ZPT, 16), jnp.float32)
    zero32 = jnp.zeros((ZPT, 32), jnp.float32)
    batch3 = batch.astype(jnp.int32).reshape(NBLK, NB, 1)

    def v2(a):
        return a.reshape(1, -1)

    a0, a1 = _sc_agg(src_p, dst_p, zero16, (x16,), 1, 16, True)
    u, st = _tc_mm1_l1(x16, a0, a1, _pad_rows(p["c11"]["w1"], 16),
                       v2(p["c11"]["b1"]))
    v, st = _tc_mm2(u, st, v2(p["c11"]["g1"]), v2(p["c11"]["be1"]),
                    p["c11"]["w2"], v2(p["c11"]["b2"]))
    oc = _tc_chunks(v, st, v2(p["c11"]["g2"]), v2(p["c11"]["be2"]))

    ag = _sc_agg(src_p, dst_p, zero32, tuple(oc), 2, 32, False)
    u, st = _tc_mm1_chunks(oc, ag, p["c12"]["w1"], v2(p["c12"]["b1"]))
    v, st = _tc_mm2(u, st, v2(p["c12"]["g1"]), v2(p["c12"]["be1"]),
                    p["c12"]["w2"], v2(p["c12"]["b2"]))
    x1c = _tc_resid1(v, st, v2(p["c12"]["g2"]), v2(p["c12"]["be2"]), x16,
                     _pad_rows(p["r1"]["w"], 16), v2(p["r1"]["b"]))

    ag = _sc_agg(src_p, dst_p, zero32, tuple(x1c), 2, 32, False)
    u, st = _tc_mm1_chunks(x1c, ag, p["c21"]["w1"], v2(p["c21"]["b1"]))
    v, st = _tc_mm2(u, st, v2(p["c21"]["g1"]), v2(p["c21"]["be1"]),
                    p["c21"]["w2"], v2(p["c21"]["b2"]))
    oc = _tc_chunks(v, st, v2(p["c21"]["g2"]), v2(p["c21"]["be2"]))

    ag = _sc_agg(src_p, dst_p, zero32, tuple(oc), 4, 32, False)
    u, st = _tc_mm1_chunks(oc, ag, p["c22"]["w1"], v2(p["c22"]["b1"]))
    v, st = _tc_mm2(u, st, v2(p["c22"]["g1"]), v2(p["c22"]["be1"]),
                    p["c22"]["w2"], v2(p["c22"]["b2"]))

    return _tc_final(v, st, v2(p["c22"]["g2"]), v2(p["c22"]["be2"]),
                     x1c[0], x1c[1], p["r2"]["w"], v2(p["r2"]["b"]), batch3)


# speedup vs baseline: 12.3092x; 1.0483x over previous
import functools

import jax
import jax.numpy as jnp
from jax import lax
from jax.experimental import pallas as pl
from jax.experimental.pallas import tpu as pltpu
from jax.experimental.pallas import tpu_sc as plsc

N = 50000
E = 1600000
G = 64

KSB = 16
EPAD = 1638400
EB = EPAD // 128
GROUPS = EB // KSB
RING = 6
LAG = 3
SPAD = 50304
ZPT = SPAD // 16
OPT = 3128


@functools.lru_cache(maxsize=None)
def _build_agg(d, split):
    F = 16 if split else 32
    nchunks = 1 if split else d // 32
    if split:
        out_type = (jax.ShapeDtypeStruct((N, F), jnp.float32),
                    jax.ShapeDtypeStruct((N, F), jnp.float32))
        n_out = 2
    else:
        out_type = (jax.ShapeDtypeStruct((nchunks, N, F), jnp.float32),)
        n_out = 1
    mesh = plsc.VectorSubcoreMesh(core_axis_name="c", subcore_axis_name="s")
    scratch = [
        pltpu.VMEM_SHARED((SPAD, F), jnp.float32),
        pltpu.VMEM((2 * KSB, 128), jnp.int32),
        pltpu.VMEM((RING, 128, F), jnp.float32),
        pltpu.SemaphoreType.DMA,
        pltpu.SemaphoreType.DMA,
    ]

    @functools.partial(
        pl.kernel, out_type=out_type, mesh=mesh, scratch_types=scratch,
        compiler_params=pltpu.CompilerParams(use_tc_tiling_on_sc=False))
    def agg(cidx, zero_hbm, xc, *rest):
        outs = rest[:n_out]
        spmem, cbuf, rbuf, gsem, ssem = rest[n_out:]
        cid = lax.axis_index("c")
        sid = lax.axis_index("s")

        def run(x_ref, out_ref, base_g, n_groups):
            pltpu.sync_copy(zero_hbm, spmem.at[pl.ds(sid * ZPT, ZPT)])
            plsc.subcore_barrier()

            def body(g, carry):
                pltpu.sync_copy(cidx.at[base_g + g], cbuf)

                def gather(j):
                    return pltpu.async_copy(
                        x_ref.at[cbuf.at[j]], rbuf.at[j % RING], gsem)

                def scatter(j):
                    return pltpu.async_copy(
                        rbuf.at[j % RING], spmem.at[cbuf.at[KSB + j]], ssem,
                        add=True)

                gd = {}
                sd = {}
                for j in range(KSB):
                    if j >= RING:
                        sd[j - RING].wait()
                    gd[j] = gather(j)
                    if j >= LAG:
                        gd[j - LAG].wait()
                        sd[j - LAG] = scatter(j - LAG)
                for j in range(KSB - LAG, KSB):
                    gd[j].wait()
                    sd[j] = scatter(j)
                for j in range(KSB - RING, KSB):
                    sd[j].wait()
                return carry

            lax.fori_loop(0, n_groups, body, 0)
            plsc.subcore_barrier()

            @pl.when(sid < 15)
            def _():
                pltpu.sync_copy(
                    spmem.at[pl.ds(sid * OPT, OPT)],
                    out_ref.at[pl.ds(sid * OPT, OPT)])

            @pl.when(sid == 15)
            def _():
                pltpu.sync_copy(
                    spmem.at[pl.ds(15 * OPT, N - 15 * OPT)],
                    out_ref.at[pl.ds(15 * OPT, N - 15 * OPT)])

            plsc.subcore_barrier()

        if split:
            wid = cid * 16 + sid
            gpw = GROUPS // 32
            for c in range(2):
                @pl.when(cid == c)
                def _(c=c):
                    run(xc.at[0], outs[c], wid * gpw, gpw)
        else:
            gpt = GROUPS // 16
            for p in range(nchunks // 2):
                for c in range(2):
                    @pl.when(cid == c)
                    def _(p=p, c=c):
                        k = 2 * p + c
                        run(xc.at[k], outs[0].at[k], sid * gpt, gpt)

    return agg


def _sc_agg(cidx, zero, xc, d, split):
    fn = _build_agg(d, bool(split))
    return fn(cidx, zero, xc)


NB = 2000
NBLK = N // NB


def _row_spec(d):
    return pl.BlockSpec((NB, d), lambda i: (i, 0))


def _full_spec(shape):
    nd = len(shape)
    return pl.BlockSpec(shape, lambda i: (0,) * nd)


def _acc_stats(u, st_ref):
    i = pl.program_id(0)
    s = jnp.sum(u, axis=0, keepdims=True)
    s2 = jnp.sum(u * u, axis=0, keepdims=True)

    @pl.when(i == 0)
    def _():
        st_ref[0:1, :] = s
        st_ref[1:2, :] = s2

    @pl.when(i != 0)
    def _():
        st_ref[0:1, :] = st_ref[0:1, :] + s
        st_ref[1:2, :] = st_ref[1:2, :] + s2


def _bnrelu(u, st, g, be):
    m = st[0:1, :] / N
    var = st[1:2, :] / N - m * m
    rs = lax.rsqrt(var + 1e-5)
    return jnp.maximum((u - m) * (rs * g) + be, 0.0)


def _mm(a, w):
    return jnp.dot(a, w, preferred_element_type=jnp.float32)


def _tc_mm1_l1(x16, p0, p1, w, b):
    do = w.shape[1]

    def body(x_ref, a_ref, b_ref, w_ref, bias_ref, u_ref, st_ref):
        pre = x_ref[...] + a_ref[...] + b_ref[...]
        u = _mm(pre, w_ref[...]) + bias_ref[...]
        u_ref[...] = u
        _acc_stats(u, st_ref)

    return pl.pallas_call(
        body, grid=(NBLK,),
        in_specs=[_row_spec(16), _row_spec(16), _row_spec(16),
                  _full_spec(w.shape), _full_spec(b.shape)],
        out_specs=[_row_spec(do), _full_spec((2, do))],
        out_shape=[jax.ShapeDtypeStruct((N, do), jnp.float32),
                   jax.ShapeDtypeStruct((2, do), jnp.float32)],
    )(x16, p0, p1, w, b)


def _chunk_spec(K):
    return pl.BlockSpec((K, NB, 32), lambda i: (0, i, 0))


def _tc_mm1(xc, ag, w, b):
    K = xc.shape[0]
    do = w.shape[1]

    def body(x_ref, a_ref, w_ref, bias_ref, u_ref, st_ref):
        pre = jnp.concatenate(
            [x_ref[k] + a_ref[k] for k in range(K)], axis=1)
        u = _mm(pre, w_ref[...]) + bias_ref[...]
        u_ref[...] = u
        _acc_stats(u, st_ref)

    return pl.pallas_call(
        body, grid=(NBLK,),
        in_specs=[_chunk_spec(K), _chunk_spec(K), _full_spec(w.shape),
                  _full_spec(b.shape)],
        out_specs=[_row_spec(do), _full_spec((2, do))],
        out_shape=[jax.ShapeDtypeStruct((N, do), jnp.float32),
                   jax.ShapeDtypeStruct((2, do), jnp.float32)],
    )(xc, ag, w, b)


def _tc_mm2(u, st, g, be, w, b):
    dh = u.shape[1]
    do = w.shape[1]

    def body(u_ref, st_ref, g_ref, be_ref, w_ref, bias_ref, v_ref, ost_ref):
        h = _bnrelu(u_ref[...], st_ref[...], g_ref[...], be_ref[...])
        v = _mm(h, w_ref[...]) + bias_ref[...]
        v_ref[...] = v
        _acc_stats(v, ost_ref)

    return pl.pallas_call(
        body, grid=(NBLK,),
        in_specs=[_row_spec(dh), _full_spec((2, dh)), _full_spec((1, dh)),
                  _full_spec((1, dh)), _full_spec(w.shape),
                  _full_spec(b.shape)],
        out_specs=[_row_spec(do), _full_spec((2, do))],
        out_shape=[jax.ShapeDtypeStruct((N, do), jnp.float32),
                   jax.ShapeDtypeStruct((2, do), jnp.float32)],
    )(u, st, g, be, w, b)


def _tc_bnrelu(v, st, g, be):
    dh = v.shape[1]
    K = dh // 32

    def body(v_ref, st_ref, g_ref, be_ref, o_ref):
        h = _bnrelu(v_ref[...], st_ref[...], g_ref[...], be_ref[...])
        for k in range(K):
            o_ref[k] = h[:, 32 * k:32 * k + 32]

    return pl.pallas_call(
        body, grid=(NBLK,),
        in_specs=[_row_spec(dh), _full_spec((2, dh)), _full_spec((1, dh)),
                  _full_spec((1, dh))],
        out_specs=_chunk_spec(K),
        out_shape=jax.ShapeDtypeStruct((K, N, 32), jnp.float32),
    )(v, st, g, be)


def _tc_resid1(v, st, g, be, x16, rw, rb):
    dh = v.shape[1]
    K = dh // 32

    def body(v_ref, st_ref, g_ref, be_ref, x_ref, rw_ref, rb_ref, o_ref):
        h = _bnrelu(v_ref[...], st_ref[...], g_ref[...], be_ref[...])
        idp = _mm(x_ref[...], rw_ref[...]) + rb_ref[...]
        x1 = jnp.maximum(h + idp, 0.0)
        for k in range(K):
            o_ref[k] = x1[:, 32 * k:32 * k + 32]

    return pl.pallas_call(
        body, grid=(NBLK,),
        in_specs=[_row_spec(dh), _full_spec((2, dh)), _full_spec((1, dh)),
                  _full_spec((1, dh)), _row_spec(16), _full_spec(rw.shape),
                  _full_spec(rb.shape)],
        out_specs=_chunk_spec(K),
        out_shape=jax.ShapeDtypeStruct((K, N, 32), jnp.float32),
    )(v, st, g, be, x16, rw, rb)


def _tc_final(v, st, g, be, x1, rw, rb, batch3):
    dh = v.shape[1]

    def body(v_ref, st_ref, g_ref, be_ref, x1_ref, rw_ref, rb_ref,
             bid_ref, out_ref):
        i = pl.program_id(0)
        h = _bnrelu(v_ref[...], st_ref[...], g_ref[...], be_ref[...])
        x1 = jnp.concatenate([x1_ref[0], x1_ref[1]], axis=1)
        idp = _mm(x1, rw_ref[...]) + rb_ref[...]
        x2 = jnp.maximum(h + idp, 0.0)

        @pl.when(i == 0)
        def _():
            out_ref[...] = jnp.full((G, dh), -jnp.inf, jnp.float32)

        bid = bid_ref[0, :, :]
        glo = bid[0, 0]
        ghi = bid[NB - 1, 0]

        def gbody(gg, carry):
            mask = bid == gg
            vals = jnp.where(mask, x2, -jnp.inf)
            cm = jnp.max(vals, axis=0, keepdims=True)
            out_ref[pl.ds(gg, 1), :] = jnp.maximum(out_ref[pl.ds(gg, 1), :],
                                                   cm)
            return carry

        lax.fori_loop(glo, ghi + 1, gbody, 0)

    return pl.pallas_call(
        body, grid=(NBLK,),
        in_specs=[_row_spec(dh), _full_spec((2, dh)), _full_spec((1, dh)),
                  _full_spec((1, dh)), _chunk_spec(2),
                  _full_spec(rw.shape), _full_spec(rb.shape),
                  pl.BlockSpec((1, NB, 1), lambda i: (i, 0, 0))],
        out_specs=_full_spec((G, dh)),
        out_shape=jax.ShapeDtypeStruct((G, dh), jnp.float32),
    )(v, st, g, be, x1, rw, rb, batch3)


def _pad_rows(w, rows):
    return jnp.pad(w, ((0, rows - w.shape[0]), (0, 0)))


def kernel(x, edge_index, batch, params):
    p = params
    src = edge_index[0].astype(jnp.int32)
    dst = edge_index[1].astype(jnp.int32)

    pad = EPAD - E
    fill = (jnp.arange(pad, dtype=jnp.int32) % 256)
    src_g = jnp.concatenate([src, fill]).reshape(GROUPS, KSB, 128)
    dst_g = jnp.concatenate([dst, N + fill]).reshape(GROUPS, KSB, 128)
    cidx = jnp.concatenate([src_g, dst_g], axis=1)

    x16 = jnp.pad(x, ((0, 0), (0, 16 - x.shape[1])))
    zero16 = jnp.zeros((ZPT, 16), jnp.float32)
    zero32 = jnp.zeros((ZPT, 32), jnp.float32)
    batch3 = batch.astype(jnp.int32).reshape(NBLK, NB, 1)

    def v2(a):
        return a.reshape(1, -1)

    a0, a1 = _sc_agg(cidx, zero16, x16.reshape(1, N, 16), 16, True)
    u, st = _tc_mm1_l1(x16, a0, a1, _pad_rows(p["c11"]["w1"], 16),
                       v2(p["c11"]["b1"]))
    v, st = _tc_mm2(u, st, v2(p["c11"]["g1"]), v2(p["c11"]["be1"]),
                    p["c11"]["w2"], v2(p["c11"]["b2"]))
    o1 = _tc_bnrelu(v, st, v2(p["c11"]["g2"]), v2(p["c11"]["be2"]))

    (ag,) = _sc_agg(cidx, zero32, o1, 64, False)
    u, st = _tc_mm1(o1, ag, p["c12"]["w1"], v2(p["c12"]["b1"]))
    v, st = _tc_mm2(u, st, v2(p["c12"]["g1"]), v2(p["c12"]["be1"]),
                    p["c12"]["w2"], v2(p["c12"]["b2"]))
    x1 = _tc_resid1(v, st, v2(p["c12"]["g2"]), v2(p["c12"]["be2"]), x16,
                    _pad_rows(p["r1"]["w"], 16), v2(p["r1"]["b"]))

    (ag,) = _sc_agg(cidx, zero32, x1, 64, False)
    u, st = _tc_mm1(x1, ag, p["c21"]["w1"], v2(p["c21"]["b1"]))
    v, st = _tc_mm2(u, st, v2(p["c21"]["g1"]), v2(p["c21"]["be1"]),
                    p["c21"]["w2"], v2(p["c21"]["b2"]))
    o2 = _tc_bnrelu(v, st, v2(p["c21"]["g2"]), v2(p["c21"]["be2"]))

    (ag,) = _sc_agg(cidx, zero32, o2, 128, False)
    u, st = _tc_mm1(o2, ag, p["c22"]["w1"], v2(p["c22"]["b1"]))
    v, st = _tc_mm2(u, st, v2(p["c22"]["g1"]), v2(p["c22"]["be1"]),
                    p["c22"]["w2"], v2(p["c22"]["b2"]))

    return _tc_final(v, st, v2(p["c22"]["g2"]), v2(p["c22"]["be2"]),
                     x1, p["r2"]["w"], v2(p["r2"]["b"]), batch3)


# speedup vs baseline: 12.7767x; 1.0380x over previous
import functools

import jax
import jax.numpy as jnp
from jax import lax
from jax.experimental import pallas as pl
from jax.experimental.pallas import tpu as pltpu
from jax.experimental.pallas import tpu_sc as plsc

N = 50000
E = 1600000
G = 64

KSB = 16
EPAD = 1638400
EB = EPAD // 128
GROUPS = EB // KSB
RING = 6
LAG = 4
SPAD = 50304
ZPT = SPAD // 16
OPT = 3128


@functools.lru_cache(maxsize=None)
def _build_agg(d, split):
    F = 16 if split else 32
    nchunks = 1 if split else d // 32
    if split:
        out_type = (jax.ShapeDtypeStruct((N, F), jnp.float32),
                    jax.ShapeDtypeStruct((N, F), jnp.float32))
        n_out = 2
    else:
        out_type = (jax.ShapeDtypeStruct((nchunks, N, F), jnp.float32),)
        n_out = 1
    mesh = plsc.VectorSubcoreMesh(core_axis_name="c", subcore_axis_name="s")
    scratch = [
        pltpu.VMEM_SHARED((SPAD, F), jnp.float32),
        pltpu.VMEM((2 * KSB, 128), jnp.int32),
        pltpu.VMEM((RING, 128, F), jnp.float32),
        pltpu.SemaphoreType.DMA,
        pltpu.SemaphoreType.DMA,
    ]

    @functools.partial(
        pl.kernel, out_type=out_type, mesh=mesh, scratch_types=scratch,
        compiler_params=pltpu.CompilerParams(use_tc_tiling_on_sc=False))
    def agg(cidx, zero_hbm, xc, *rest):
        outs = rest[:n_out]
        spmem, cbuf, rbuf, gsem, ssem = rest[n_out:]
        cid = lax.axis_index("c")
        sid = lax.axis_index("s")

        def run(x_ref, out_ref, base_g, n_groups):
            pltpu.sync_copy(zero_hbm, spmem.at[pl.ds(sid * ZPT, ZPT)])
            plsc.subcore_barrier()

            def body(g, carry):
                pltpu.sync_copy(cidx.at[base_g + g], cbuf)

                def gather(j):
                    return pltpu.async_copy(
                        x_ref.at[cbuf.at[j]], rbuf.at[j % RING], gsem)

                def scatter(j):
                    return pltpu.async_copy(
                        rbuf.at[j % RING], spmem.at[cbuf.at[KSB + j]], ssem,
                        add=True)

                gd = {}
                sd = {}
                for j in range(KSB):
                    if j >= RING:
                        sd[j - RING].wait()
                    gd[j] = gather(j)
                    if j >= LAG:
                        gd[j - LAG].wait()
                        sd[j - LAG] = scatter(j - LAG)
                for j in range(KSB - LAG, KSB):
                    gd[j].wait()
                    sd[j] = scatter(j)
                for j in range(KSB - RING, KSB):
                    sd[j].wait()
                return carry

            lax.fori_loop(0, n_groups, body, 0)
            plsc.subcore_barrier()

            @pl.when(sid < 15)
            def _():
                pltpu.sync_copy(
                    spmem.at[pl.ds(sid * OPT, OPT)],
                    out_ref.at[pl.ds(sid * OPT, OPT)])

            @pl.when(sid == 15)
            def _():
                pltpu.sync_copy(
                    spmem.at[pl.ds(15 * OPT, N - 15 * OPT)],
                    out_ref.at[pl.ds(15 * OPT, N - 15 * OPT)])

            plsc.subcore_barrier()

        if split:
            wid = cid * 16 + sid
            gpw = GROUPS // 32
            for c in range(2):
                @pl.when(cid == c)
                def _(c=c):
                    run(xc.at[0], outs[c], wid * gpw, gpw)
        else:
            gpt = GROUPS // 16
            for p in range(nchunks // 2):
                for c in range(2):
                    @pl.when(cid == c)
                    def _(p=p, c=c):
                        k = 2 * p + c
                        run(xc.at[k], outs[0].at[k], sid * gpt, gpt)

    return agg


def _sc_agg(cidx, zero, xc, d, split):
    fn = _build_agg(d, bool(split))
    return fn(cidx, zero, xc)


NB = 2000
NBLK = N // NB


def _row_spec(d):
    return pl.BlockSpec((NB, d), lambda i: (i, 0))


def _full_spec(shape):
    nd = len(shape)
    return pl.BlockSpec(shape, lambda i: (0,) * nd)


def _acc_stats(u, st_ref):
    i = pl.program_id(0)
    s = jnp.sum(u, axis=0, keepdims=True)
    s2 = jnp.sum(u * u, axis=0, keepdims=True)

    @pl.when(i == 0)
    def _():
        st_ref[0:1, :] = s
        st_ref[1:2, :] = s2

    @pl.when(i != 0)
    def _():
        st_ref[0:1, :] = st_ref[0:1, :] + s
        st_ref[1:2, :] = st_ref[1:2, :] + s2


def _bnrelu(u, st, g, be):
    m = st[0:1, :] / N
    var = st[1:2, :] / N - m * m
    rs = lax.rsqrt(var + 1e-5)
    return jnp.maximum((u - m) * (rs * g) + be, 0.0)


def _mm(a, w):
    return jnp.dot(a, w, preferred_element_type=jnp.float32)


def _tc_mm1_l1(x16, p0, p1, w, b):
    do = w.shape[1]

    def body(x_ref, a_ref, b_ref, w_ref, bias_ref, u_ref, st_ref):
        pre = x_ref[...] + a_ref[...] + b_ref[...]
        u = _mm(pre, w_ref[...]) + bias_ref[...]
        u_ref[...] = u
        _acc_stats(u, st_ref)

    return pl.pallas_call(
        body, grid=(NBLK,),
        in_specs=[_row_spec(16), _row_spec(16), _row_spec(16),
                  _full_spec(w.shape), _full_spec(b.shape)],
        out_specs=[_row_spec(do), _full_spec((2, do))],
        out_shape=[jax.ShapeDtypeStruct((N, do), jnp.float32),
                   jax.ShapeDtypeStruct((2, do), jnp.float32)],
    )(x16, p0, p1, w, b)


def _chunk_spec(K):
    return pl.BlockSpec((K, NB, 32), lambda i: (0, i, 0))


def _tc_mm1(xc, ag, w, b):
    K = xc.shape[0]
    do = w.shape[1]

    def body(x_ref, a_ref, w_ref, bias_ref, u_ref, st_ref):
        pre = jnp.concatenate(
            [x_ref[k] + a_ref[k] for k in range(K)], axis=1)
        u = _mm(pre, w_ref[...]) + bias_ref[...]
        u_ref[...] = u
        _acc_stats(u, st_ref)

    return pl.pallas_call(
        body, grid=(NBLK,),
        in_specs=[_chunk_spec(K), _chunk_spec(K), _full_spec(w.shape),
                  _full_spec(b.shape)],
        out_specs=[_row_spec(do), _full_spec((2, do))],
        out_shape=[jax.ShapeDtypeStruct((N, do), jnp.float32),
                   jax.ShapeDtypeStruct((2, do), jnp.float32)],
    )(xc, ag, w, b)


def _tc_mm2(u, st, g, be, w, b):
    dh = u.shape[1]
    do = w.shape[1]

    def body(u_ref, st_ref, g_ref, be_ref, w_ref, bias_ref, v_ref, ost_ref):
        h = _bnrelu(u_ref[...], st_ref[...], g_ref[...], be_ref[...])
        v = _mm(h, w_ref[...]) + bias_ref[...]
        v_ref[...] = v
        _acc_stats(v, ost_ref)

    return pl.pallas_call(
        body, grid=(NBLK,),
        in_specs=[_row_spec(dh), _full_spec((2, dh)), _full_spec((1, dh)),
                  _full_spec((1, dh)), _full_spec(w.shape),
                  _full_spec(b.shape)],
        out_specs=[_row_spec(do), _full_spec((2, do))],
        out_shape=[jax.ShapeDtypeStruct((N, do), jnp.float32),
                   jax.ShapeDtypeStruct((2, do), jnp.float32)],
    )(u, st, g, be, w, b)


def _tc_bnrelu(v, st, g, be):
    dh = v.shape[1]
    K = dh // 32

    def body(v_ref, st_ref, g_ref, be_ref, o_ref):
        h = _bnrelu(v_ref[...], st_ref[...], g_ref[...], be_ref[...])
        for k in range(K):
            o_ref[k] = lax.reshape(lax.reshape(h[:, 32 * k:32 * k + 32],
                                               (NB * 32,)), (NB, 32))

    return pl.pallas_call(
        body, grid=(NBLK,),
        in_specs=[_row_spec(dh), _full_spec((2, dh)), _full_spec((1, dh)),
                  _full_spec((1, dh))],
        out_specs=_chunk_spec(K),
        out_shape=jax.ShapeDtypeStruct((K, N, 32), jnp.float32),
    )(v, st, g, be)


def _tc_resid1(v, st, g, be, x16, rw, rb):
    dh = v.shape[1]
    K = dh // 32

    def body(v_ref, st_ref, g_ref, be_ref, x_ref, rw_ref, rb_ref, o_ref):
        h = _bnrelu(v_ref[...], st_ref[...], g_ref[...], be_ref[...])
        idp = _mm(x_ref[...], rw_ref[...]) + rb_ref[...]
        x1 = jnp.maximum(h + idp, 0.0)
        for k in range(K):
            o_ref[k] = x1[:, 32 * k:32 * k + 32]

    return pl.pallas_call(
        body, grid=(NBLK,),
        in_specs=[_row_spec(dh), _full_spec((2, dh)), _full_spec((1, dh)),
                  _full_spec((1, dh)), _row_spec(16), _full_spec(rw.shape),
                  _full_spec(rb.shape)],
        out_specs=_chunk_spec(K),
        out_shape=jax.ShapeDtypeStruct((K, N, 32), jnp.float32),
    )(v, st, g, be, x16, rw, rb)


def _tc_final(v, st, g, be, x1, rw, rb, batch3):
    dh = v.shape[1]

    def body(v_ref, st_ref, g_ref, be_ref, x1_ref, rw_ref, rb_ref,
             bid_ref, out_ref):
        i = pl.program_id(0)
        h = _bnrelu(v_ref[...], st_ref[...], g_ref[...], be_ref[...])
        x1 = jnp.concatenate([x1_ref[0], x1_ref[1]], axis=1)
        idp = _mm(x1, rw_ref[...]) + rb_ref[...]
        x2 = jnp.maximum(h + idp, 0.0)

        @pl.when(i == 0)
        def _():
            out_ref[...] = jnp.full((G, dh), -jnp.inf, jnp.float32)

        bid = bid_ref[0, :, :]
        glo = bid[0, 0]
        ghi = bid[NB - 1, 0]

        def gbody(gg, carry):
            mask = bid == gg
            vals = jnp.where(mask, x2, -jnp.inf)
            cm = jnp.max(vals, axis=0, keepdims=True)
            out_ref[pl.ds(gg, 1), :] = jnp.maximum(out_ref[pl.ds(gg, 1), :],
                                                   cm)
            return carry

        lax.fori_loop(glo, ghi + 1, gbody, 0)

    return pl.pallas_call(
        body, grid=(NBLK,),
        in_specs=[_row_spec(dh), _full_spec((2, dh)), _full_spec((1, dh)),
                  _full_spec((1, dh)), _chunk_spec(2),
                  _full_spec(rw.shape), _full_spec(rb.shape),
                  pl.BlockSpec((1, NB, 1), lambda i: (i, 0, 0))],
        out_specs=_full_spec((G, dh)),
        out_shape=jax.ShapeDtypeStruct((G, dh), jnp.float32),
    )(v, st, g, be, x1, rw, rb, batch3)


def _pad_rows(w, rows):
    return jnp.pad(w, ((0, rows - w.shape[0]), (0, 0)))


def kernel(x, edge_index, batch, params):
    p = params
    src = edge_index[0].astype(jnp.int32)
    dst = edge_index[1].astype(jnp.int32)

    pad = EPAD - E
    fill = (jnp.arange(pad, dtype=jnp.int32) % 256)
    src_g = jnp.concatenate([src, fill]).reshape(GROUPS, KSB, 128)
    dst_g = jnp.concatenate([dst, N + fill]).reshape(GROUPS, KSB, 128)
    cidx = jnp.concatenate([src_g, dst_g], axis=1)

    x16 = jnp.pad(x, ((0, 0), (0, 16 - x.shape[1])))
    zero16 = jnp.zeros((ZPT, 16), jnp.float32)
    zero32 = jnp.zeros((ZPT, 32), jnp.float32)
    batch3 = batch.astype(jnp.int32).reshape(NBLK, NB, 1)

    def v2(a):
        return a.reshape(1, -1)

    a0, a1 = _sc_agg(cidx, zero16, x16.reshape(1, N, 16), 16, True)
    u, st = _tc_mm1_l1(x16, a0, a1, _pad_rows(p["c11"]["w1"], 16),
                       v2(p["c11"]["b1"]))
    v, st = _tc_mm2(u, st, v2(p["c11"]["g1"]), v2(p["c11"]["be1"]),
                    p["c11"]["w2"], v2(p["c11"]["b2"]))
    o1 = _tc_bnrelu(v, st, v2(p["c11"]["g2"]), v2(p["c11"]["be2"]))

    (ag,) = _sc_agg(cidx, zero32, o1, 64, False)
    u, st = _tc_mm1(o1, ag, p["c12"]["w1"], v2(p["c12"]["b1"]))
    v, st = _tc_mm2(u, st, v2(p["c12"]["g1"]), v2(p["c12"]["be1"]),
                    p["c12"]["w2"], v2(p["c12"]["b2"]))
    x1 = _tc_resid1(v, st, v2(p["c12"]["g2"]), v2(p["c12"]["be2"]), x16,
                    _pad_rows(p["r1"]["w"], 16), v2(p["r1"]["b"]))

    (ag,) = _sc_agg(cidx, zero32, x1, 64, False)
    u, st = _tc_mm1(x1, ag, p["c21"]["w1"], v2(p["c21"]["b1"]))
    v, st = _tc_mm2(u, st, v2(p["c21"]["g1"]), v2(p["c21"]["be1"]),
                    p["c21"]["w2"], v2(p["c21"]["b2"]))
    o2 = _tc_bnrelu(v, st, v2(p["c21"]["g2"]), v2(p["c21"]["be2"]))

    (ag,) = _sc_agg(cidx, zero32, o2, 128, False)
    u, st = _tc_mm1(o2, ag, p["c22"]["w1"], v2(p["c22"]["b1"]))
    v, st = _tc_mm2(u, st, v2(p["c22"]["g1"]), v2(p["c22"]["be1"]),
                    p["c22"]["w2"], v2(p["c22"]["b2"]))

    return _tc_final(v, st, v2(p["c22"]["g2"]), v2(p["c22"]["be2"]),
                     x1, p["r2"]["w"], v2(p["r2"]["b"]), batch3)
